# baseline (speedup 1.0000x reference)
import jax
import jax.numpy as jnp
from jax import lax
from jax.experimental import pallas as pl
from jax.experimental.pallas import tpu as pltpu
from jax.experimental.pallas import tpu_sc as plsc

EMB = 128
BATCH = 16384

_INFO = plsc.get_sparse_core_info()
NC = _INFO.num_cores
NS = _INFO.num_subcores
L = _INFO.num_lanes
NW = NC * NS
ROWS_PER_W = BATCH // NW
CHUNK = 128
NCHUNK = ROWS_PER_W // CHUNK
NGRP = CHUNK // L
NBUF = 3


def _lane_shuffle(x, idx):
    dnums = lax.GatherDimensionNumbers(
        offset_dims=(), collapsed_slice_dims=(0,), start_index_map=(0,))
    return lax.gather(x, idx[:, None], dnums, (1,),
                      mode=lax.GatherScatterMode.PROMISE_IN_BOUNDS)


def _body(u_hbm, v_hbm, user_hbm, item_hbm, out_hbm,
          uidx_v, vidx_v, ue_v, ve_v, out_v, sem0, sem1, sem2):
    wid = lax.axis_index("s") * NC + lax.axis_index("c")
    base = wid * ROWS_PER_W

    lanes = lax.iota(jnp.int32, L)
    sems = (sem0, sem1, sem2)

    pltpu.sync_copy(u_hbm.at[pl.ds(base, ROWS_PER_W)], uidx_v)
    pltpu.sync_copy(v_hbm.at[pl.ds(base, ROWS_PER_W)], vidx_v)

    def fire(c):
        b = c % NBUF
        cp_u = pltpu.make_async_copy(
            user_hbm.at[uidx_v.at[pl.ds(c * CHUNK, CHUNK)]], ue_v.at[b],
            sems[b])
        cp_v = pltpu.make_async_copy(
            item_hbm.at[vidx_v.at[pl.ds(c * CHUNK, CHUNK)]], ve_v.at[b],
            sems[b])
        cp_u.start()
        cp_v.start()
        return cp_u, cp_v

    inflight = [fire(c) for c in range(NBUF - 1)]
    for c in range(NCHUNK):
        if c + NBUF - 1 < NCHUNK:
            inflight.append(fire(c + NBUF - 1))
        cp_u, cp_v = inflight.pop(0)
        cp_u.wait()
        cp_v.wait()
        b = c % NBUF

        def group_body(g, _):
            ubuf = ue_v.at[b]
            vbuf = ve_v.at[b]

            def row_body(r, vec):
                urow = ubuf.at[g * L + r]
                vrow = vbuf.at[g * L + r]
                prods = [urow[pl.ds(k * L, L)] * vrow[pl.ds(k * L, L)]
                         for k in range(EMB // L)]
                while len(prods) > 1:
                    prods = [prods[i] + prods[i + 1]
                             for i in range(0, len(prods), 2)]
                acc = prods[0]
                for sh in (8, 4, 2, 1):
                    acc = acc + _lane_shuffle(acc, lanes ^ sh)
                return jnp.where(lanes == r, acc, vec)

            vec = lax.fori_loop(0, L, row_body, jnp.zeros((L,), jnp.float32),
                                unroll=2)
            out_v[pl.ds(c * CHUNK + g * L, L)] = vec
            return 0

        lax.fori_loop(0, NGRP, group_body, 0)

    pltpu.sync_copy(out_v, out_hbm.at[pl.ds(base, ROWS_PER_W)])


@jax.jit
def kernel(u, v, user_emb, item_emb):
    mesh = plsc.VectorSubcoreMesh(core_axis_name="c", subcore_axis_name="s")
    run = pl.kernel(
        _body,
        mesh=mesh,
        out_type=jax.ShapeDtypeStruct((BATCH,), jnp.float32),
        scratch_types=[
            pltpu.VMEM((ROWS_PER_W,), jnp.int32),
            pltpu.VMEM((ROWS_PER_W,), jnp.int32),
            pltpu.VMEM((NBUF, CHUNK, EMB), jnp.float32),
            pltpu.VMEM((NBUF, CHUNK, EMB), jnp.float32),
            pltpu.VMEM((ROWS_PER_W,), jnp.float32),
            pltpu.SemaphoreType.DMA,
            pltpu.SemaphoreType.DMA,
            pltpu.SemaphoreType.DMA,
        ],
    )
    return run(u, v, user_emb, item_emb)


# speedup vs baseline: 1.0137x; 1.0137x over previous
import jax
import jax.numpy as jnp
from jax import lax
from jax.experimental import pallas as pl
from jax.experimental.pallas import tpu as pltpu
from jax.experimental.pallas import tpu_sc as plsc

EMB = 128
BATCH = 16384

_INFO = plsc.get_sparse_core_info()
NC = _INFO.num_cores
NS = _INFO.num_subcores
L = _INFO.num_lanes
NW = NC * NS
ROWS_PER_W = BATCH // NW
CHUNK = 64
NCHUNK = ROWS_PER_W // CHUNK
NGRP = CHUNK // L
NBUF = 4


def _lane_shuffle(x, idx):
    dnums = lax.GatherDimensionNumbers(
        offset_dims=(), collapsed_slice_dims=(0,), start_index_map=(0,))
    return lax.gather(x, idx[:, None], dnums, (1,),
                      mode=lax.GatherScatterMode.PROMISE_IN_BOUNDS)


def _body(u_hbm, v_hbm, user_hbm, item_hbm, out_hbm,
          uidx_v, vidx_v, ue_v, ve_v, out_v, sem0, sem1, sem2, sem3):
    wid = lax.axis_index("s") * NC + lax.axis_index("c")
    base = wid * ROWS_PER_W

    lanes = lax.iota(jnp.int32, L)
    sems = (sem0, sem1, sem2, sem3)

    pltpu.sync_copy(u_hbm.at[pl.ds(base, ROWS_PER_W)], uidx_v)
    pltpu.sync_copy(v_hbm.at[pl.ds(base, ROWS_PER_W)], vidx_v)

    def fire(c):
        b = c % NBUF
        cp_u = pltpu.make_async_copy(
            user_hbm.at[uidx_v.at[pl.ds(c * CHUNK, CHUNK)]], ue_v.at[b],
            sems[b])
        cp_v = pltpu.make_async_copy(
            item_hbm.at[vidx_v.at[pl.ds(c * CHUNK, CHUNK)]], ve_v.at[b],
            sems[b])
        cp_u.start()
        cp_v.start()
        return cp_u, cp_v

    inflight = [fire(c) for c in range(NBUF - 1)]
    for c in range(NCHUNK):
        if c + NBUF - 1 < NCHUNK:
            inflight.append(fire(c + NBUF - 1))
        cp_u, cp_v = inflight.pop(0)
        cp_u.wait()
        cp_v.wait()
        b = c % NBUF

        def group_body(g, _):
            ubuf = ue_v.at[b]
            vbuf = ve_v.at[b]

            def row_body(r, vec):
                urow = ubuf.at[g * L + r]
                vrow = vbuf.at[g * L + r]
                prods = [urow[pl.ds(k * L, L)] * vrow[pl.ds(k * L, L)]
                         for k in range(EMB // L)]
                while len(prods) > 1:
                    prods = [prods[i] + prods[i + 1]
                             for i in range(0, len(prods), 2)]
                acc = prods[0]
                for sh in (8, 4, 2, 1):
                    acc = acc + _lane_shuffle(acc, lanes ^ sh)
                return jnp.where(lanes == r, acc, vec)

            vec = lax.fori_loop(0, L, row_body, jnp.zeros((L,), jnp.float32),
                                unroll=2)
            out_v[pl.ds(c * CHUNK + g * L, L)] = vec
            return 0

        lax.fori_loop(0, NGRP, group_body, 0)

    pltpu.sync_copy(out_v, out_hbm.at[pl.ds(base, ROWS_PER_W)])


@jax.jit
def kernel(u, v, user_emb, item_emb):
    mesh = plsc.VectorSubcoreMesh(core_axis_name="c", subcore_axis_name="s")
    run = pl.kernel(
        _body,
        mesh=mesh,
        out_type=jax.ShapeDtypeStruct((BATCH,), jnp.float32),
        scratch_types=[
            pltpu.VMEM((ROWS_PER_W,), jnp.int32),
            pltpu.VMEM((ROWS_PER_W,), jnp.int32),
            pltpu.VMEM((NBUF, CHUNK, EMB), jnp.float32),
            pltpu.VMEM((NBUF, CHUNK, EMB), jnp.float32),
            pltpu.VMEM((ROWS_PER_W,), jnp.float32),
            pltpu.SemaphoreType.DMA,
            pltpu.SemaphoreType.DMA,
            pltpu.SemaphoreType.DMA,
            pltpu.SemaphoreType.DMA,
        ],
    )
    return run(u, v, user_emb, item_emb)


# speedup vs baseline: 1.0368x; 1.0228x over previous
import jax
import jax.numpy as jnp
from jax import lax
from jax.experimental import pallas as pl
from jax.experimental.pallas import tpu as pltpu
from jax.experimental.pallas import tpu_sc as plsc

EMB = 128
BATCH = 16384

_INFO = plsc.get_sparse_core_info()
NC = _INFO.num_cores
NS = _INFO.num_subcores
L = _INFO.num_lanes
NW = NC * NS
ROWS_PER_W = BATCH // NW
CHUNK = 64
NCHUNK = ROWS_PER_W // CHUNK
NGRP = CHUNK // L
NBUF = 4


def _lane_shuffle(x, idx):
    dnums = lax.GatherDimensionNumbers(
        offset_dims=(), collapsed_slice_dims=(0,), start_index_map=(0,))
    return lax.gather(x, idx[:, None], dnums, (1,),
                      mode=lax.GatherScatterMode.PROMISE_IN_BOUNDS)


def _body(u_hbm, v_hbm, user_hbm, item_hbm, out_hbm,
          uidx_v, vidx_v, ue_v, ve_v, out_v,
          sem0, sem1, sem2, sem3, sem_idx, sem_out):
    wid = lax.axis_index("s") * NC + lax.axis_index("c")
    base = wid * ROWS_PER_W

    lanes = lax.iota(jnp.int32, L)
    sems = (sem0, sem1, sem2, sem3)

    cpi0u = pltpu.make_async_copy(
        u_hbm.at[pl.ds(base, CHUNK)], uidx_v.at[pl.ds(0, CHUNK)], sem_idx)
    cpi0v = pltpu.make_async_copy(
        v_hbm.at[pl.ds(base, CHUNK)], vidx_v.at[pl.ds(0, CHUNK)], sem_idx)
    cpi0u.start()
    cpi0v.start()
    cpi0u.wait()
    cpi0v.wait()

    def fire(c):
        b = c % NBUF
        cp_u = pltpu.make_async_copy(
            user_hbm.at[uidx_v.at[pl.ds(c * CHUNK, CHUNK)]], ue_v.at[b],
            sems[b])
        cp_v = pltpu.make_async_copy(
            item_hbm.at[vidx_v.at[pl.ds(c * CHUNK, CHUNK)]], ve_v.at[b],
            sems[b])
        cp_u.start()
        cp_v.start()
        return cp_u, cp_v

    inflight = [fire(0)]

    rest = ROWS_PER_W - CHUNK
    cpiru = pltpu.make_async_copy(
        u_hbm.at[pl.ds(base + CHUNK, rest)], uidx_v.at[pl.ds(CHUNK, rest)],
        sem_idx)
    cpirv = pltpu.make_async_copy(
        v_hbm.at[pl.ds(base + CHUNK, rest)], vidx_v.at[pl.ds(CHUNK, rest)],
        sem_idx)
    cpiru.start()
    cpirv.start()
    cpiru.wait()
    cpirv.wait()

    for c in range(1, NBUF - 1):
        inflight.append(fire(c))

    out_cps = []
    for c in range(NCHUNK):
        if c + NBUF - 1 < NCHUNK:
            inflight.append(fire(c + NBUF - 1))
        cp_u, cp_v = inflight.pop(0)
        cp_u.wait()
        cp_v.wait()
        b = c % NBUF

        def group_body(g, _):
            ubuf = ue_v.at[b]
            vbuf = ve_v.at[b]

            def row_body(r, vec):
                urow = ubuf.at[g * L + r]
                vrow = vbuf.at[g * L + r]
                prods = [urow[pl.ds(k * L, L)] * vrow[pl.ds(k * L, L)]
                         for k in range(EMB // L)]
                while len(prods) > 1:
                    prods = [prods[i] + prods[i + 1]
                             for i in range(0, len(prods), 2)]
                acc = prods[0]
                for sh in (8, 4, 2, 1):
                    acc = acc + _lane_shuffle(acc, lanes ^ sh)
                return jnp.where(lanes == r, acc, vec)

            vec = lax.fori_loop(0, L, row_body, jnp.zeros((L,), jnp.float32),
                                unroll=2)
            out_v[pl.ds(c * CHUNK + g * L, L)] = vec
            return 0

        lax.fori_loop(0, NGRP, group_body, 0)

        cp_o = pltpu.make_async_copy(
            out_v.at[pl.ds(c * CHUNK, CHUNK)],
            out_hbm.at[pl.ds(base + c * CHUNK, CHUNK)], sem_out)
        cp_o.start()
        out_cps.append(cp_o)

    for cp_o in out_cps:
        cp_o.wait()


@jax.jit
def kernel(u, v, user_emb, item_emb):
    mesh = plsc.VectorSubcoreMesh(core_axis_name="c", subcore_axis_name="s")
    run = pl.kernel(
        _body,
        mesh=mesh,
        out_type=jax.ShapeDtypeStruct((BATCH,), jnp.float32),
        scratch_types=[
            pltpu.VMEM((ROWS_PER_W,), jnp.int32),
            pltpu.VMEM((ROWS_PER_W,), jnp.int32),
            pltpu.VMEM((NBUF, CHUNK, EMB), jnp.float32),
            pltpu.VMEM((NBUF, CHUNK, EMB), jnp.float32),
            pltpu.VMEM((ROWS_PER_W,), jnp.float32),
            pltpu.SemaphoreType.DMA,
            pltpu.SemaphoreType.DMA,
            pltpu.SemaphoreType.DMA,
            pltpu.SemaphoreType.DMA,
            pltpu.SemaphoreType.DMA,
            pltpu.SemaphoreType.DMA,
        ],
    )
    return run(u, v, user_emb, item_emb)
